# baseline (speedup 1.0000x reference)
import functools

import jax
import jax.numpy as jnp
from jax import lax
from jax.experimental import pallas as pl
from jax.experimental.pallas import tpu as pltpu
from jax.experimental.pallas import tpu_sc as plsc

N = 10000
E = 160000
D = 256
_RB = 2000

_NC, _NS = 2, 16
_NW = _NC * _NS
_RW = 313
_NPAD = _NW * _RW
_CE = 4000
_NCHUNK = E // _CE
_VPC = _CE // 16
_G = 128
_MCAP = _CE + 256
_PK = 512
_FMAX = 3.0e38


def _pool_body(x_ref, w_ref, b_ref, o_ref):
    acc = jax.lax.dot_general(
        x_ref[...], w_ref[...], (((1,), (1,)), ((), ())),
        preferred_element_type=jnp.float32)
    o_ref[...] = jnp.maximum(acc + b_ref[...], 0.0).astype(jnp.bfloat16)


def _pool_matmul(x, W, b):
    return pl.pallas_call(
        _pool_body,
        grid=(N // _RB,),
        in_specs=[
            pl.BlockSpec((_RB, D), lambda i: (i, 0)),
            pl.BlockSpec((D, D), lambda i: (0, 0)),
            pl.BlockSpec((1, D), lambda i: (0, 0)),
        ],
        out_specs=pl.BlockSpec((_RB, D), lambda i: (i, 0)),
        out_shape=jax.ShapeDtypeStruct((N, D), jnp.bfloat16),
    )(x, W, b.reshape(1, D))


def _out_body(x_ref, ws_ref, e_ref, we_ref, b_ref, o_ref, *, relu):
    acc = jax.lax.dot_general(
        x_ref[...], ws_ref[...], (((1,), (1,)), ((), ())),
        preferred_element_type=jnp.float32)
    acc = acc + jax.lax.dot_general(
        e_ref[...], we_ref[...], (((1,), (1,)), ((), ())),
        preferred_element_type=jnp.float32)
    acc = acc + b_ref[...]
    if relu:
        acc = jnp.maximum(acc, 0.0)
    o_ref[...] = acc


def _out_matmul(x, Wself, eve, Weve, b, relu):
    return pl.pallas_call(
        functools.partial(_out_body, relu=relu),
        grid=(N // _RB,),
        in_specs=[
            pl.BlockSpec((_RB, D), lambda i: (i, 0)),
            pl.BlockSpec((D, D), lambda i: (0, 0)),
            pl.BlockSpec((_RB, D), lambda i: (i, 0)),
            pl.BlockSpec((D, D), lambda i: (0, 0)),
            pl.BlockSpec((1, D), lambda i: (0, 0)),
        ],
        out_specs=pl.BlockSpec((_RB, D), lambda i: (i, 0)),
        out_shape=jax.ShapeDtypeStruct((N, D), jnp.float32),
    )(x, Wself, eve, Weve, b.reshape(1, D))


def _sc_eve_body(h_hbm, src_hbm, dst_hbm, w_hbm, out_hbm,
                 amax, amin, dstb, srcb, mlist, idxb, rows, wr, sem):
    wid = lax.axis_index("s") * _NC + lax.axis_index("c")
    lo = wid * _RW

    pltpu.sync_copy(w_hbm, wr)
    w0 = wr[pl.ds(0, 32)]
    w1 = wr[pl.ds(32, 32)]
    wb = wr[pl.ds(64, 32)]

    def _inita(i, _):
        amax[pl.ds(i * 32, 32)] = jnp.full((32,), -1.0, jnp.bfloat16)
        amin[pl.ds(i * 32, 32)] = jnp.full((32,), _FMAX, jnp.bfloat16)
        return 0
    lax.fori_loop(0, (_RW + 1) * D // 32, _inita, 0)

    trash = jax.lax.iota(jnp.int32, 16) + (_MCAP - 16)
    trashval = jnp.full((16,), _RW, jnp.int32)
    lov = jnp.broadcast_to(lo, (16,)).astype(jnp.int32)
    hiv = lov + _RW

    def _chunk(c, _):
        pltpu.sync_copy(dst_hbm.at[pl.ds(c * _CE, _CE)], dstb)
        pltpu.sync_copy(src_hbm.at[pl.ds(c * _CE, _CE)], srcb)

        def _scan(v, cnt):
            dvec = dstb[pl.ds(v * 16, 16)]
            svec = srcb[pl.ds(v * 16, 16)]
            m = (dvec >= lov) & (dvec < hiv)
            mi = m.astype(jnp.int32)
            cs = plsc.cumsum(mi)
            cntv = jnp.broadcast_to(cnt, (16,)).astype(jnp.int32)
            pos = jnp.where(m, cntv + cs - mi, trash)
            plsc.store_scatter(mlist, [pos], svec * _PK + (dvec - lov))
            return cnt + cs[15]

        cnt = lax.fori_loop(0, _VPC, _scan, jnp.int32(0))
        for t in range(_G // 16):
            mlist[pl.ds(cnt + t * 16, 16)] = trashval
        nb = (cnt + _G - 1) // _G

        def _batch(b, _):
            for t in range(_G // 16):
                pk = mlist[pl.ds(b * _G + t * 16, 16)]
                idxb[pl.ds(t * 16, 16)] = pk // _PK
            pltpu.async_copy(h_hbm.at[idxb], rows, sem).wait()

            def _edge(j, _):
                pk = mlist[pl.ds(b * _G + j, 16)][0]
                base = (pk % _PK) * D
                for k in range(D // 32):
                    rv = plsc.bitcast(rows[j, pl.ds(k * 16, 16)],
                                      jnp.bfloat16)
                    off = base + k * 32
                    amax[pl.ds(off, 32)] = jnp.maximum(
                        amax[pl.ds(off, 32)], rv)
                    amin[pl.ds(off, 32)] = jnp.minimum(
                        amin[pl.ds(off, 32)], rv)
                return 0
            lax.fori_loop(0, _G, _edge, 0)
            return 0

        lax.fori_loop(0, nb, _batch, 0)
        return 0

    lax.fori_loop(0, _NCHUNK, _chunk, 0)

    zero = jnp.zeros((32,), jnp.bfloat16)

    def _fin(i, _):
        off = i * 32
        mx = amax[pl.ds(off, 32)]
        mn = amin[pl.ds(off, 32)]
        ne = mx < zero
        mx = jnp.where(ne, zero, mx)
        mn = jnp.where(ne, zero, mn)
        amax[pl.ds(off, 32)] = jnp.maximum(w0 * mx + w1 * mn + wb, zero)
        return 0
    lax.fori_loop(0, _RW * D // 32, _fin, 0)
    pltpu.sync_copy(amax.at[pl.ds(0, _RW * D)],
                    out_hbm.at[pl.ds(lo * D, _RW * D)])


def _sc_eve(h, src, dst, dww, dwb):
    w = jnp.concatenate([jnp.full((32,), dww[0], jnp.float32),
                         jnp.full((32,), dww[1], jnp.float32),
                         jnp.full((32,), dwb[0], jnp.float32)]
                        ).astype(jnp.bfloat16)
    mesh = plsc.VectorSubcoreMesh(core_axis_name="c", subcore_axis_name="s",
                                  num_cores=_NC, num_subcores=_NS)
    run = pl.kernel(
        _sc_eve_body,
        out_type=jax.ShapeDtypeStruct((_NPAD * D,), jnp.bfloat16),
        mesh=mesh,
        scratch_types=[
            pltpu.VMEM(((_RW + 1) * D,), jnp.bfloat16),
            pltpu.VMEM(((_RW + 1) * D,), jnp.bfloat16),
            pltpu.VMEM((_CE,), jnp.int32),
            pltpu.VMEM((_CE,), jnp.int32),
            pltpu.VMEM((_MCAP,), jnp.int32),
            pltpu.VMEM((_G,), jnp.int32),
            pltpu.VMEM((_G, D // 2), jnp.int32),
            pltpu.VMEM((96,), jnp.bfloat16),
            pltpu.SemaphoreType.DMA,
        ],
        compiler_params=pltpu.CompilerParams(needs_layout_passes=False),
    )
    eve = run(h, src, dst, w)
    return eve.reshape(_NPAD, D)[:N]


def _layer(x, src, dst, Wpool, bpool, dww, dwb, Weve, Wself, bias, relu):
    h = _pool_matmul(x, Wpool, bpool)
    hp = jax.lax.bitcast_convert_type(h.reshape(N, D // 2, 2), jnp.int32)
    eve = _sc_eve(hp, src, dst, dww, dwb)
    return _out_matmul(x, Wself, eve, Weve.astype(jnp.bfloat16), bias, relu)


def kernel(x, edge_index, c1_Wpool, c1_bpool, c1_dww, c1_dwb, c1_Weve, c1_Wself, c1_bias, c2_Wpool, c2_bpool, c2_dww, c2_dwb, c2_Weve, c2_Wself, c2_bias):
    src = edge_index[0]
    dst = edge_index[1]
    h = _layer(x, src, dst, c1_Wpool, c1_bpool, c1_dww, c1_dwb, c1_Weve,
               c1_Wself, c1_bias, relu=True)
    return _layer(h, src, dst, c2_Wpool, c2_bpool, c2_dww, c2_dwb, c2_Weve,
                  c2_Wself, c2_bias, relu=False)
